# baseline (speedup 1.0000x reference)
import functools

import jax
import jax.numpy as jnp
from jax import lax
from jax.experimental import pallas as pl
from jax.experimental.pallas import tpu as pltpu
from jax.experimental.pallas import tpu_sc as plsc

_N_FIELDS = 26
_VOCAB = 100000
_EMB = 32
_BATCH = 16384
_NUM = 13
_OUT_D = _N_FIELDS * _EMB + _NUM
_SLOTS = 53

_NC = 2
_NS = 16
_NW = _NC * _NS

_ROWS = _BATCH * _N_FIELDS
_B_PER_W = _BATCH // _NW
_ROWS_PER_W = _B_PER_W * _N_FIELDS
_SUB = 128
_NSUB_W = _ROWS_PER_W // _SUB
_B_CHUNK = 64
_CHUNK = _B_CHUNK * _N_FIELDS
_NSUB = _CHUNK // _SUB
_NCHUNK = _B_PER_W // _B_CHUNK

_VCHUNK = 2176
_NVCHUNK = -(-_VOCAB // _VCHUNK)


def _transpose_body(t_ref, o_ref):
    x = t_ref[0]
    x3 = x.reshape(_EMB, _VCHUNK // 4, 4)
    z3 = jnp.transpose(x3, (1, 2, 0))
    o_ref[0] = z3.reshape(_VCHUNK // 4, 128)


def _tc_transpose(tables_t):
    return pl.pallas_call(
        _transpose_body,
        grid=(_N_FIELDS, _NVCHUNK),
        in_specs=[
            pl.BlockSpec((1, _EMB, _VCHUNK), lambda f, j: (f, 0, j)),
        ],
        out_specs=pl.BlockSpec(
            (1, _VCHUNK // 4, 128), lambda f, j: (f, j, 0)
        ),
        out_shape=jax.ShapeDtypeStruct(
            (_N_FIELDS, _VOCAB // 4, 128), jnp.float32
        ),
        compiler_params=pltpu.CompilerParams(
            dimension_semantics=("parallel", "arbitrary")
        ),
    )(tables_t)


_cache = {}


def _build_kernel():
    if "k" in _cache:
        return _cache["k"]
    mesh = plsc.VectorSubcoreMesh(core_axis_name="c", subcore_axis_name="s")

    @functools.partial(
        pl.kernel,
        mesh=mesh,
        out_type=jax.ShapeDtypeStruct((_BATCH, _SLOTS * 16), jnp.float32),
        compiler_params=pltpu.CompilerParams(
            use_tc_tiling_on_sc=False, needs_layout_passes=False
        ),
        scratch_types=[
            pltpu.VMEM((_NSUB_W, _SUB), jnp.int32),
            pltpu.VMEM((_CHUNK, _EMB), jnp.float32),
            pltpu.VMEM((_B_CHUNK, _SLOTS * 16), jnp.float32),
            pltpu.VMEM((_B_PER_W, 16), jnp.float32),
            pltpu.VMEM((_B_PER_W,), jnp.float32),
            pltpu.SemaphoreType.DMA,
            pltpu.SemaphoreType.DMA,
        ],
    )
    def _k(idx_hbm, table_hbm, num_hbm, scale_hbm, out_hbm,
           idx_v, ebuf, cb, num_v, scale_v, sem, wsem):
        wid = lax.axis_index("s") * _NC + lax.axis_index("c")
        b0w = wid * _B_PER_W

        pltpu.sync_copy(idx_hbm.at[pl.ds(wid * _NSUB_W, _NSUB_W)], idx_v)
        pltpu.sync_copy(num_hbm.at[pl.ds(b0w, _B_PER_W)], num_v)
        pltpu.sync_copy(scale_hbm.at[pl.ds(b0w, _B_PER_W)], scale_v)

        def chunk_body(ci, carry):
            copies = []
            for j in range(_NSUB):
                copies.append(
                    pltpu.async_copy(
                        table_hbm.at[idx_v.at[ci * _NSUB + j]],
                        ebuf.at[pl.ds(j * _SUB, _SUB)],
                        sem,
                    )
                )
            for c in copies:
                c.wait()

            def row_body(k, c2):
                s = plsc.load_gather(
                    scale_v, [jnp.full((16,), ci * _B_CHUNK + k, jnp.int32)]
                )
                r0 = k * _N_FIELDS
                for f in range(_N_FIELDS):
                    cb[k, pl.ds(32 * f, 16)] = ebuf[r0 + f, pl.ds(0, 16)] * s
                    cb[k, pl.ds(32 * f + 16, 16)] = (
                        ebuf[r0 + f, pl.ds(16, 16)] * s
                    )
                cb[k, pl.ds(16 * (_SLOTS - 1), 16)] = num_v[
                    ci * _B_CHUNK + k, :
                ]
                return c2

            lax.fori_loop(0, _B_CHUNK, row_body, 0)

            wcp = pltpu.async_copy(
                cb, out_hbm.at[pl.ds(b0w + ci * _B_CHUNK, _B_CHUNK)], wsem
            )
            wcp.wait()
            return carry

        lax.fori_loop(0, _NCHUNK, chunk_body, 0)

    _cache["k"] = _k
    return _k


def kernel(numeric_features, categorical_features, mask, tables):
    cat = categorical_features.astype(jnp.int32)
    offs = (jnp.arange(_N_FIELDS, dtype=jnp.int32) * _VOCAB)[None, :]
    idx = (cat + offs).reshape(_ROWS // _SUB, _SUB)
    tables_t = jnp.swapaxes(tables, 1, 2)
    t128 = _tc_transpose(tables_t)
    table2d = t128.reshape(_N_FIELDS * _VOCAB, _EMB)
    num_pad = jnp.pad(numeric_features, ((0, 0), (0, 16 - _NUM)))
    scale = jnp.where(mask, 0.0, 1.0).astype(jnp.float32).reshape(_BATCH)
    padded = _build_kernel()(idx, table2d, num_pad, scale)
    return padded[:, :_OUT_D]


# speedup vs baseline: 4.4115x; 4.4115x over previous
import functools

import jax
import jax.numpy as jnp
from jax import lax
from jax.experimental import pallas as pl
from jax.experimental.pallas import tpu as pltpu
from jax.experimental.pallas import tpu_sc as plsc

_N_FIELDS = 26
_VOCAB = 100000
_EMB = 32
_BATCH = 16384
_NUM = 13
_OUT_D = _N_FIELDS * _EMB + _NUM
_SLOTS = 53

_NC = 2
_NS = 16
_NW = _NC * _NS

_ROWS = _BATCH * _N_FIELDS
_B_PER_W = _BATCH // _NW
_ROWS_PER_W = _B_PER_W * _N_FIELDS
_SUB = 128
_NSUB_W = _ROWS_PER_W // _SUB
_B_CHUNK = 64
_CHUNK = _B_CHUNK * _N_FIELDS
_NSUB = _CHUNK // _SUB
_NCHUNK = _B_PER_W // _B_CHUNK

_VCHUNK = 2176
_NVCHUNK = -(-_VOCAB // _VCHUNK)


def _transpose_body(t_ref, o_ref):
    x = t_ref[0]
    z = x.T
    z3 = z.reshape(_VCHUNK // 4, 4, _EMB)
    o_ref[0] = jnp.concatenate([z3[:, s, :] for s in range(4)], axis=1)


def _tc_transpose(tables_t):
    return pl.pallas_call(
        _transpose_body,
        grid=(_N_FIELDS, _NVCHUNK),
        in_specs=[
            pl.BlockSpec((1, _EMB, _VCHUNK), lambda f, j: (f, 0, j)),
        ],
        out_specs=pl.BlockSpec(
            (1, _VCHUNK // 4, 128), lambda f, j: (f, j, 0)
        ),
        out_shape=jax.ShapeDtypeStruct(
            (_N_FIELDS, _VOCAB // 4, 128), jnp.float32
        ),
        compiler_params=pltpu.CompilerParams(
            dimension_semantics=("parallel", "arbitrary")
        ),
    )(tables_t)


_cache = {}


def _build_kernel():
    if "k" in _cache:
        return _cache["k"]
    mesh = plsc.VectorSubcoreMesh(core_axis_name="c", subcore_axis_name="s")

    @functools.partial(
        pl.kernel,
        mesh=mesh,
        out_type=jax.ShapeDtypeStruct((_BATCH, _SLOTS * 16), jnp.float32),
        compiler_params=pltpu.CompilerParams(
            use_tc_tiling_on_sc=False, needs_layout_passes=False
        ),
        scratch_types=[
            pltpu.VMEM((_NSUB_W, _SUB), jnp.int32),
            pltpu.VMEM((_CHUNK, _EMB), jnp.float32),
            pltpu.VMEM((_B_CHUNK, _SLOTS * 16), jnp.float32),
            pltpu.VMEM((_B_PER_W, 16), jnp.float32),
            pltpu.VMEM((_B_PER_W,), jnp.float32),
            pltpu.SemaphoreType.DMA,
            pltpu.SemaphoreType.DMA,
        ],
    )
    def _k(idx_hbm, table_hbm, num_hbm, scale_hbm, out_hbm,
           idx_v, ebuf, cb, num_v, scale_v, sem, wsem):
        wid = lax.axis_index("s") * _NC + lax.axis_index("c")
        b0w = wid * _B_PER_W

        pltpu.sync_copy(idx_hbm.at[pl.ds(wid * _NSUB_W, _NSUB_W)], idx_v)
        pltpu.sync_copy(num_hbm.at[pl.ds(b0w, _B_PER_W)], num_v)
        pltpu.sync_copy(scale_hbm.at[pl.ds(b0w, _B_PER_W)], scale_v)

        def chunk_body(ci, carry):
            copies = []
            for j in range(_NSUB):
                copies.append(
                    pltpu.async_copy(
                        table_hbm.at[idx_v.at[ci * _NSUB + j]],
                        ebuf.at[pl.ds(j * _SUB, _SUB)],
                        sem,
                    )
                )
            for c in copies:
                c.wait()

            def row_body(k, c2):
                s = plsc.load_gather(
                    scale_v, [jnp.full((16,), ci * _B_CHUNK + k, jnp.int32)]
                )
                r0 = k * _N_FIELDS
                for f in range(_N_FIELDS):
                    cb[k, pl.ds(32 * f, 16)] = ebuf[r0 + f, pl.ds(0, 16)] * s
                    cb[k, pl.ds(32 * f + 16, 16)] = (
                        ebuf[r0 + f, pl.ds(16, 16)] * s
                    )
                cb[k, pl.ds(16 * (_SLOTS - 1), 16)] = num_v[
                    ci * _B_CHUNK + k, :
                ]
                return c2

            lax.fori_loop(0, _B_CHUNK, row_body, 0)

            wcp = pltpu.async_copy(
                cb, out_hbm.at[pl.ds(b0w + ci * _B_CHUNK, _B_CHUNK)], wsem
            )
            wcp.wait()
            return carry

        lax.fori_loop(0, _NCHUNK, chunk_body, 0)

    _cache["k"] = _k
    return _k


def kernel(numeric_features, categorical_features, mask, tables):
    cat = categorical_features.astype(jnp.int32)
    offs = (jnp.arange(_N_FIELDS, dtype=jnp.int32) * _VOCAB)[None, :]
    idx = (cat + offs).reshape(_ROWS // _SUB, _SUB)
    tables_t = jnp.swapaxes(tables, 1, 2)
    t128 = _tc_transpose(tables_t)
    table2d = t128.reshape(_N_FIELDS * _VOCAB, _EMB)
    num_pad = jnp.pad(numeric_features, ((0, 0), (0, 16 - _NUM)))
    scale = jnp.where(mask, 0.0, 1.0).astype(jnp.float32).reshape(_BATCH)
    padded = _build_kernel()(idx, table2d, num_pad, scale)
    return padded[:, :_OUT_D]


# speedup vs baseline: 4.6777x; 1.0603x over previous
import functools

import jax
import jax.numpy as jnp
from jax import lax
from jax.experimental import pallas as pl
from jax.experimental.pallas import tpu as pltpu
from jax.experimental.pallas import tpu_sc as plsc

_N_FIELDS = 26
_VOCAB = 100000
_EMB = 32
_BATCH = 16384
_NUM = 13
_OUT_D = _N_FIELDS * _EMB + _NUM
_SLOTS = 53

_NC = 2
_NS = 16
_NW = _NC * _NS

_ROWS = _BATCH * _N_FIELDS
_B_PER_W = _BATCH // _NW
_ROWS_PER_W = _B_PER_W * _N_FIELDS
_SUB = 128
_NSUB_W = _ROWS_PER_W // _SUB
_B_CHUNK = 64
_CHUNK = _B_CHUNK * _N_FIELDS
_NSUB = _CHUNK // _SUB
_NCHUNK = _B_PER_W // _B_CHUNK

_VCHUNK = 8704
_NVCHUNK = -(-_VOCAB // _VCHUNK)


def _transpose_body(t_ref, o_ref):
    x = t_ref[0]
    eye = jax.lax.broadcasted_iota(jnp.int32, (_EMB, _EMB), 0) == (
        jax.lax.broadcasted_iota(jnp.int32, (_EMB, _EMB), 1)
    )
    z = jax.lax.dot_general(
        x,
        eye.astype(jnp.float32),
        (((0,), (0,)), ((), ())),
        preferred_element_type=jnp.float32,
    )
    z3 = z.reshape(_VCHUNK // 4, 4, _EMB)
    o_ref[0] = jnp.concatenate([z3[:, s, :] for s in range(4)], axis=1)


def _tc_transpose(tables_t):
    return pl.pallas_call(
        _transpose_body,
        grid=(_N_FIELDS, _NVCHUNK),
        in_specs=[
            pl.BlockSpec((1, _EMB, _VCHUNK), lambda f, j: (f, 0, j)),
        ],
        out_specs=pl.BlockSpec(
            (1, _VCHUNK // 4, 128), lambda f, j: (f, j, 0)
        ),
        out_shape=jax.ShapeDtypeStruct(
            (_N_FIELDS, _VOCAB // 4, 128), jnp.float32
        ),
        compiler_params=pltpu.CompilerParams(
            dimension_semantics=("parallel", "arbitrary")
        ),
    )(tables_t)


_cache = {}


def _build_kernel():
    if "k" in _cache:
        return _cache["k"]
    mesh = plsc.VectorSubcoreMesh(core_axis_name="c", subcore_axis_name="s")

    @functools.partial(
        pl.kernel,
        mesh=mesh,
        out_type=jax.ShapeDtypeStruct((_BATCH, _SLOTS * 16), jnp.float32),
        compiler_params=pltpu.CompilerParams(
            use_tc_tiling_on_sc=False, needs_layout_passes=False
        ),
        scratch_types=[
            pltpu.VMEM((_NSUB_W, _SUB), jnp.int32),
            pltpu.VMEM((_CHUNK, _EMB), jnp.float32),
            pltpu.VMEM((_B_CHUNK, _SLOTS * 16), jnp.float32),
            pltpu.VMEM((_B_PER_W, 16), jnp.float32),
            pltpu.VMEM((_B_PER_W,), jnp.float32),
            pltpu.SemaphoreType.DMA,
            pltpu.SemaphoreType.DMA,
        ],
    )
    def _k(idx_hbm, table_hbm, num_hbm, scale_hbm, out_hbm,
           idx_v, ebuf, cb, num_v, scale_v, sem, wsem):
        wid = lax.axis_index("s") * _NC + lax.axis_index("c")
        b0w = wid * _B_PER_W

        pltpu.sync_copy(idx_hbm.at[pl.ds(wid * _NSUB_W, _NSUB_W)], idx_v)
        pltpu.sync_copy(num_hbm.at[pl.ds(b0w, _B_PER_W)], num_v)
        pltpu.sync_copy(scale_hbm.at[pl.ds(b0w, _B_PER_W)], scale_v)

        def chunk_body(ci, carry):
            copies = []
            for j in range(_NSUB):
                copies.append(
                    pltpu.async_copy(
                        table_hbm.at[idx_v.at[ci * _NSUB + j]],
                        ebuf.at[pl.ds(j * _SUB, _SUB)],
                        sem,
                    )
                )
            for c in copies:
                c.wait()

            def row_body(k, c2):
                s = plsc.load_gather(
                    scale_v, [jnp.full((16,), ci * _B_CHUNK + k, jnp.int32)]
                )
                r0 = k * _N_FIELDS
                for f in range(_N_FIELDS):
                    cb[k, pl.ds(32 * f, 16)] = ebuf[r0 + f, pl.ds(0, 16)] * s
                    cb[k, pl.ds(32 * f + 16, 16)] = (
                        ebuf[r0 + f, pl.ds(16, 16)] * s
                    )
                cb[k, pl.ds(16 * (_SLOTS - 1), 16)] = num_v[
                    ci * _B_CHUNK + k, :
                ]
                return c2

            lax.fori_loop(0, _B_CHUNK, row_body, 0)

            wcp = pltpu.async_copy(
                cb, out_hbm.at[pl.ds(b0w + ci * _B_CHUNK, _B_CHUNK)], wsem
            )
            wcp.wait()
            return carry

        lax.fori_loop(0, _NCHUNK, chunk_body, 0)

    _cache["k"] = _k
    return _k


def kernel(numeric_features, categorical_features, mask, tables):
    cat = categorical_features.astype(jnp.int32)
    offs = (jnp.arange(_N_FIELDS, dtype=jnp.int32) * _VOCAB)[None, :]
    idx = (cat + offs).reshape(_ROWS // _SUB, _SUB)
    tables_t = jnp.swapaxes(tables, 1, 2)
    t128 = _tc_transpose(tables_t)
    table2d = t128.reshape(_N_FIELDS * _VOCAB, _EMB)
    num_pad = jnp.pad(numeric_features, ((0, 0), (0, 16 - _NUM)))
    scale = jnp.where(mask, 0.0, 1.0).astype(jnp.float32).reshape(_BATCH)
    padded = _build_kernel()(idx, table2d, num_pad, scale)
    return padded[:, :_OUT_D]


# speedup vs baseline: 6.7517x; 1.4434x over previous
import functools

import jax
import jax.numpy as jnp
from jax import lax
from jax.experimental import pallas as pl
from jax.experimental.pallas import tpu as pltpu
from jax.experimental.pallas import tpu_sc as plsc

_N_FIELDS = 26
_VOCAB = 100000
_EMB = 32
_BATCH = 16384
_NUM = 13
_OUT_D = _N_FIELDS * _EMB + _NUM
_SLOTS = 53

_NC = 2
_NS = 16
_NW = _NC * _NS

_ROWS = _BATCH * _N_FIELDS
_B_PER_W = _BATCH // _NW
_ROWS_PER_W = _B_PER_W * _N_FIELDS
_SUB = 128
_NSUB_W = _ROWS_PER_W // _SUB
_B_CHUNK = 64
_CHUNK = _B_CHUNK * _N_FIELDS
_NSUB = _CHUNK // _SUB
_NCHUNK = _B_PER_W // _B_CHUNK

_VCHUNK = 8704
_NVCHUNK = -(-_VOCAB // _VCHUNK)


def _transpose_body(t_ref, o_ref):
    x = t_ref[0]
    eye = jax.lax.broadcasted_iota(jnp.int32, (_EMB, _EMB), 0) == (
        jax.lax.broadcasted_iota(jnp.int32, (_EMB, _EMB), 1)
    )
    z = jax.lax.dot_general(
        x,
        eye.astype(jnp.float32),
        (((0,), (0,)), ((), ())),
        preferred_element_type=jnp.float32,
    )
    z5 = z.reshape(_VCHUNK // 32, 4, 8, _EMB)
    o_ref[0] = jnp.concatenate([z5[:, s] for s in range(4)], axis=2)


def _tc_transpose(tables_t):
    return pl.pallas_call(
        _transpose_body,
        grid=(_N_FIELDS, _NVCHUNK),
        in_specs=[
            pl.BlockSpec((1, _EMB, _VCHUNK), lambda f, j: (f, 0, j)),
        ],
        out_specs=pl.BlockSpec(
            (1, _VCHUNK // 32, 8, 128), lambda f, j: (f, j, 0, 0)
        ),
        out_shape=jax.ShapeDtypeStruct(
            (_N_FIELDS, _VOCAB // 32, 8, 128), jnp.float32
        ),
        compiler_params=pltpu.CompilerParams(
            dimension_semantics=("parallel", "arbitrary")
        ),
    )(tables_t)


_cache = {}


def _build_kernel():
    if "k" in _cache:
        return _cache["k"]
    mesh = plsc.VectorSubcoreMesh(core_axis_name="c", subcore_axis_name="s")

    @functools.partial(
        pl.kernel,
        mesh=mesh,
        out_type=jax.ShapeDtypeStruct((_BATCH, _SLOTS * 16), jnp.float32),
        compiler_params=pltpu.CompilerParams(
            use_tc_tiling_on_sc=False, needs_layout_passes=False
        ),
        scratch_types=[
            pltpu.VMEM((_NSUB_W, _SUB), jnp.int32),
            pltpu.VMEM((_CHUNK, _EMB), jnp.float32),
            pltpu.VMEM((_B_CHUNK, _SLOTS * 16), jnp.float32),
            pltpu.VMEM((_B_PER_W, 16), jnp.float32),
            pltpu.VMEM((_B_PER_W,), jnp.float32),
            pltpu.SemaphoreType.DMA,
            pltpu.SemaphoreType.DMA,
        ],
    )
    def _k(idx_hbm, table_hbm, num_hbm, scale_hbm, out_hbm,
           idx_v, ebuf, cb, num_v, scale_v, sem, wsem):
        wid = lax.axis_index("s") * _NC + lax.axis_index("c")
        b0w = wid * _B_PER_W

        pltpu.sync_copy(idx_hbm.at[pl.ds(wid * _NSUB_W, _NSUB_W)], idx_v)
        pltpu.sync_copy(num_hbm.at[pl.ds(b0w, _B_PER_W)], num_v)
        pltpu.sync_copy(scale_hbm.at[pl.ds(b0w, _B_PER_W)], scale_v)

        def chunk_body(ci, carry):
            copies = []
            for j in range(_NSUB):
                copies.append(
                    pltpu.async_copy(
                        table_hbm.at[idx_v.at[ci * _NSUB + j]],
                        ebuf.at[pl.ds(j * _SUB, _SUB)],
                        sem,
                    )
                )
            for c in copies:
                c.wait()

            def row_body(k, c2):
                s = plsc.load_gather(
                    scale_v, [jnp.full((16,), ci * _B_CHUNK + k, jnp.int32)]
                )
                r0 = k * _N_FIELDS
                for f in range(_N_FIELDS):
                    cb[k, pl.ds(32 * f, 16)] = ebuf[r0 + f, pl.ds(0, 16)] * s
                    cb[k, pl.ds(32 * f + 16, 16)] = (
                        ebuf[r0 + f, pl.ds(16, 16)] * s
                    )
                cb[k, pl.ds(16 * (_SLOTS - 1), 16)] = num_v[
                    ci * _B_CHUNK + k, :
                ]
                return c2

            lax.fori_loop(0, _B_CHUNK, row_body, 0)

            wcp = pltpu.async_copy(
                cb, out_hbm.at[pl.ds(b0w + ci * _B_CHUNK, _B_CHUNK)], wsem
            )
            wcp.wait()
            return carry

        lax.fori_loop(0, _NCHUNK, chunk_body, 0)

    _cache["k"] = _k
    return _k


def kernel(numeric_features, categorical_features, mask, tables):
    cat = categorical_features.astype(jnp.int32)
    offs = (jnp.arange(_N_FIELDS, dtype=jnp.int32) * _VOCAB)[None, :]
    g = cat + offs
    g = (g & ~jnp.int32(31)) | ((g & 7) << 2) | ((g >> 3) & 3)
    idx = g.reshape(_ROWS // _SUB, _SUB)
    tables_t = jnp.swapaxes(tables, 1, 2)
    t128 = _tc_transpose(tables_t)
    table2d = t128.reshape(_N_FIELDS * _VOCAB, _EMB)
    num_pad = jnp.pad(numeric_features, ((0, 0), (0, 16 - _NUM)))
    scale = jnp.where(mask, 0.0, 1.0).astype(jnp.float32).reshape(_BATCH)
    padded = _build_kernel()(idx, table2d, num_pad, scale)
    return padded[:, :_OUT_D]
